# baseline (speedup 1.0000x reference)
import functools

import jax
import jax.numpy as jnp
from jax import lax
from jax.experimental import pallas as pl
from jax.experimental.pallas import tpu as pltpu
from jax.experimental.pallas import tpu_sc as plsc

D = 64
DP = 128
V = 169
NC, NS = 2, 16
NW = NC * NS
CH = 256
IR = CH // 128


@functools.partial(jax.jit, static_argnames=("n_chunks",))
def _sc_gather(table_pad, idx2d, n_chunks):
    B = n_chunks * NW * CH
    mesh = plsc.VectorSubcoreMesh(core_axis_name="c", subcore_axis_name="s")

    @functools.partial(
        pl.kernel,
        out_type=jax.ShapeDtypeStruct((B, DP), jnp.float32),
        mesh=mesh,
        scratch_types=[
            pltpu.VMEM((2, IR, 128), jnp.int32),
            pltpu.VMEM((CH, DP), jnp.float32),
            pltpu.VMEM((CH, DP), jnp.float32),
            pltpu.VMEM_SHARED((V, DP), jnp.float32),
            pltpu.SemaphoreType.DMA,
            pltpu.SemaphoreType.DMA,
            pltpu.SemaphoreType.DMA,
            pltpu.SemaphoreType.DMA,
            pltpu.SemaphoreType.DMA,
        ],
        compiler_params=pltpu.CompilerParams(use_tc_tiling_on_sc=True),
    )
    def k(table_hbm, idx_hbm, out_hbm, idx_v, rows0, rows1, table_s,
          gat_sem, idx_sem0, idx_sem1, out_sem0, out_sem1):
        rows_v = (rows0, rows1)
        idx_sem = (idx_sem0, idx_sem1)
        out_sem = (out_sem0, out_sem1)
        wid = lax.axis_index("s") * NC + lax.axis_index("c")
        sid = lax.axis_index("s")

        @pl.when(sid == 0)
        def _stage_table():
            pltpu.sync_copy(table_hbm, table_s)

        plsc.subcore_barrier()

        def irow0(i):
            return (wid * n_chunks + i) * IR

        def fire_idx(i, b):
            pltpu.async_copy(
                idx_hbm.at[pl.ds(irow0(i), IR)], idx_v.at[b], idx_sem[b])

        fire_idx(0, 0)
        fire_idx(1, 1)

        @pl.loop(0, n_chunks, step=2)
        def _chunk(g):
            for b in range(2):
                i = g + b
                pltpu.make_async_copy(
                    idx_hbm.at[pl.ds(irow0(i), IR)], idx_v.at[b],
                    idx_sem[b]).wait()

                @pl.when(g >= 2)
                def _drain_store():
                    pltpu.make_async_copy(
                        rows_v[b],
                        out_hbm.at[pl.ds((wid * n_chunks + i - 2) * CH, CH)],
                        out_sem[b]).wait()

                for j in range(IR):
                    pltpu.async_copy(
                        table_s.at[idx_v.at[b].at[j]],
                        rows_v[b].at[pl.ds(j * 128, 128)],
                        gat_sem,
                    )
                for j in range(IR):
                    pltpu.make_async_copy(
                        table_s.at[idx_v.at[b].at[j]],
                        rows_v[b].at[pl.ds(j * 128, 128)],
                        gat_sem,
                    ).wait()

                @pl.when(i + 2 < n_chunks)
                def _prefetch_idx():
                    fire_idx(i + 2, b)

                pltpu.async_copy(
                    rows_v[b],
                    out_hbm.at[pl.ds((wid * n_chunks + i) * CH, CH)],
                    out_sem[b])

        for b in range(2):
            i = n_chunks - 2 + b
            pltpu.make_async_copy(
                rows_v[b],
                out_hbm.at[pl.ds((wid * n_chunks + i) * CH, CH)],
                out_sem[b]).wait()

    return k(table_pad, idx2d)


BB = 8


@functools.partial(jax.jit, static_argnames=("n0", "n1"))
def _tc_slice(wide, n0, n1):
    def body(in_ref, out_ref):
        out_ref[...] = in_ref[:, :D].reshape(BB, n1, D)

    return pl.pallas_call(
        body,
        grid=(n0 // BB,),
        in_specs=[pl.BlockSpec((BB * n1, DP), lambda i: (i, 0))],
        out_specs=pl.BlockSpec((BB, n1, D), lambda i: (i, 0, 0)),
        out_shape=jax.ShapeDtypeStruct((n0, n1, D), jnp.float32),
    )(wide)


def kernel(x, table):
    n0, n1 = x.shape
    B = n0 * n1
    table_pad = jnp.zeros((V, DP), jnp.float32).at[:, :D].set(table)
    idx2d = x.reshape(B // 128, 128).astype(jnp.int32)
    out = _sc_gather(table_pad, idx2d, B // (NW * CH))
    return _tc_slice(out, n0, n1)


# speedup vs baseline: 1.3370x; 1.3370x over previous
import functools

import jax
import jax.numpy as jnp
from jax import lax
from jax.experimental import pallas as pl
from jax.experimental.pallas import tpu as pltpu
from jax.experimental.pallas import tpu_sc as plsc

D = 64
DP = 128
V = 169
NC, NS = 2, 16
NW = NC * NS
CH = 256
IR = CH // 128


@functools.partial(jax.jit, static_argnames=("n_chunks",))
def _sc_gather(table_pad, idx2d, n_chunks):
    B = n_chunks * NW * CH
    mesh = plsc.VectorSubcoreMesh(core_axis_name="c", subcore_axis_name="s")

    @functools.partial(
        pl.kernel,
        out_type=jax.ShapeDtypeStruct((B, DP), jnp.float32),
        mesh=mesh,
        scratch_types=[
            pltpu.VMEM((2, IR, 128), jnp.int32),
            pltpu.VMEM((CH, DP), jnp.float32),
            pltpu.VMEM((CH, DP), jnp.float32),
            pltpu.VMEM_SHARED((V, DP), jnp.float32),
            pltpu.SemaphoreType.DMA,
            pltpu.SemaphoreType.DMA,
            pltpu.SemaphoreType.DMA,
            pltpu.SemaphoreType.DMA,
            pltpu.SemaphoreType.DMA,
        ],
        compiler_params=pltpu.CompilerParams(use_tc_tiling_on_sc=True),
    )
    def k(table_hbm, idx_hbm, out_hbm, idx_v, rows0, rows1, table_s,
          gat_sem, idx_sem0, idx_sem1, out_sem0, out_sem1):
        rows_v = (rows0, rows1)
        idx_sem = (idx_sem0, idx_sem1)
        out_sem = (out_sem0, out_sem1)
        wid = lax.axis_index("s") * NC + lax.axis_index("c")
        sid = lax.axis_index("s")

        @pl.when(sid == 0)
        def _stage_table():
            pltpu.sync_copy(table_hbm, table_s)

        plsc.subcore_barrier()

        def irow0(i):
            return (wid * n_chunks + i) * IR

        def fire_idx(i, b):
            pltpu.async_copy(
                idx_hbm.at[pl.ds(irow0(i), IR)], idx_v.at[b], idx_sem[b])

        fire_idx(0, 0)
        fire_idx(1, 1)

        @pl.loop(0, n_chunks, step=2)
        def _chunk(g):
            for b in range(2):
                i = g + b
                pltpu.make_async_copy(
                    idx_hbm.at[pl.ds(irow0(i), IR)], idx_v.at[b],
                    idx_sem[b]).wait()

                @pl.when(g >= 2)
                def _drain_store():
                    pltpu.make_async_copy(
                        rows_v[b],
                        out_hbm.at[pl.ds((wid * n_chunks + i - 2) * CH, CH)],
                        out_sem[b]).wait()

                for j in range(IR):
                    pltpu.async_copy(
                        table_s.at[idx_v.at[b].at[j]],
                        rows_v[b].at[pl.ds(j * 128, 128)],
                        gat_sem,
                    )
                for j in range(IR):
                    pltpu.make_async_copy(
                        table_s.at[idx_v.at[b].at[j]],
                        rows_v[b].at[pl.ds(j * 128, 128)],
                        gat_sem,
                    ).wait()

                @pl.when(i + 2 < n_chunks)
                def _prefetch_idx():
                    fire_idx(i + 2, b)

                pltpu.async_copy(
                    rows_v[b],
                    out_hbm.at[pl.ds((wid * n_chunks + i) * CH, CH)],
                    out_sem[b])

        for b in range(2):
            i = n_chunks - 2 + b
            pltpu.make_async_copy(
                rows_v[b],
                out_hbm.at[pl.ds((wid * n_chunks + i) * CH, CH)],
                out_sem[b]).wait()

    return k(table_pad, idx2d)


NQ = 4


def kernel(x, table):
    n0, n1 = x.shape
    B = n0 * n1
    table_pad = jnp.zeros((V, DP), jnp.float32).at[:, :D].set(table)
    idx2d = x.reshape(B // 128, 128).astype(jnp.int32)
    rq = B // NQ
    pieces = []
    for q in range(NQ):
        oq = _sc_gather(table_pad,
                        lax.slice_in_dim(idx2d, q * (rq // 128),
                                         (q + 1) * (rq // 128)),
                        rq // (NW * CH))
        pieces.append(oq[:, :D])
    return jnp.concatenate(pieces, axis=0).reshape(n0, n1, D)


# speedup vs baseline: 2.5849x; 1.9334x over previous
import functools

import jax
import jax.numpy as jnp
from jax import lax
from jax.experimental import pallas as pl
from jax.experimental.pallas import tpu as pltpu
from jax.experimental.pallas import tpu_sc as plsc

D = 64
DP = 128
V = 169
NC, NS = 2, 16
NW = NC * NS
CH = 256
IR = CH // 128


@functools.partial(jax.jit, static_argnames=("n_chunks",))
def _sc_gather(table_pad, idx2d, n_chunks):
    B = n_chunks * NW * CH
    mesh = plsc.VectorSubcoreMesh(core_axis_name="c", subcore_axis_name="s")

    @functools.partial(
        pl.kernel,
        out_type=jax.ShapeDtypeStruct((B, DP), jnp.float32),
        mesh=mesh,
        scratch_types=[
            pltpu.VMEM((2, IR, 128), jnp.int32),
            pltpu.VMEM((CH, DP), jnp.float32),
            pltpu.VMEM((CH, DP), jnp.float32),
            pltpu.VMEM_SHARED((V, DP), jnp.float32),
            pltpu.SemaphoreType.DMA,
            pltpu.SemaphoreType.DMA,
            pltpu.SemaphoreType.DMA,
            pltpu.SemaphoreType.DMA,
            pltpu.SemaphoreType.DMA,
        ],
        compiler_params=pltpu.CompilerParams(use_tc_tiling_on_sc=True),
    )
    def k(table_hbm, idx_hbm, out_hbm, idx_v, rows0, rows1, table_s,
          gat_sem, idx_sem0, idx_sem1, out_sem0, out_sem1):
        rows_v = (rows0, rows1)
        idx_sem = (idx_sem0, idx_sem1)
        out_sem = (out_sem0, out_sem1)
        wid = lax.axis_index("s") * NC + lax.axis_index("c")
        sid = lax.axis_index("s")

        @pl.when(sid == 0)
        def _stage_table():
            pltpu.sync_copy(table_hbm, table_s)

        plsc.subcore_barrier()

        def irow0(i):
            return (wid * n_chunks + i) * IR

        def fire_idx(i, b):
            pltpu.async_copy(
                idx_hbm.at[pl.ds(irow0(i), IR)], idx_v.at[b], idx_sem[b])

        fire_idx(0, 0)
        fire_idx(1, 1)

        @pl.loop(0, n_chunks, step=2)
        def _chunk(g):
            for b in range(2):
                i = g + b
                pltpu.make_async_copy(
                    idx_hbm.at[pl.ds(irow0(i), IR)], idx_v.at[b],
                    idx_sem[b]).wait()

                @pl.when(g >= 2)
                def _drain_store():
                    pltpu.make_async_copy(
                        rows_v[b],
                        out_hbm.at[pl.ds((wid * n_chunks + i - 2) * CH, CH)],
                        out_sem[b]).wait()

                for j in range(IR):
                    pltpu.async_copy(
                        table_s.at[idx_v.at[b].at[j]],
                        rows_v[b].at[pl.ds(j * 128, 128)],
                        gat_sem,
                    )
                for j in range(IR):
                    pltpu.make_async_copy(
                        table_s.at[idx_v.at[b].at[j]],
                        rows_v[b].at[pl.ds(j * 128, 128)],
                        gat_sem,
                    ).wait()

                @pl.when(i + 2 < n_chunks)
                def _prefetch_idx():
                    fire_idx(i + 2, b)

                pltpu.async_copy(
                    rows_v[b],
                    out_hbm.at[pl.ds((wid * n_chunks + i) * CH, CH)],
                    out_sem[b])

        for b in range(2):
            i = n_chunks - 2 + b
            pltpu.make_async_copy(
                rows_v[b],
                out_hbm.at[pl.ds((wid * n_chunks + i) * CH, CH)],
                out_sem[b]).wait()

    return k(table_pad, idx2d)


def kernel(x, table):
    n0, n1 = x.shape
    B = n0 * n1
    table_pad = jnp.zeros((V, DP), jnp.float32).at[:, :D].set(table)
    idx2d = x.reshape(B // 128, 128).astype(jnp.int32)
    out = _sc_gather(table_pad, idx2d, B // (NW * CH))
    return out[:, :D].reshape(n0, n1, D)
